# Initial kernel scaffold; baseline (speedup 1.0000x reference)
import jax
import jax.numpy as jnp
from jax.experimental import pallas as pl


def kernel(x, edge_index, weight):
    raise NotImplementedError("write your pallas kernel here")


# speedup vs baseline: 4.4500x; 4.4500x over previous
import functools

import jax
import jax.numpy as jnp
from jax import lax
from jax.experimental import pallas as pl
from jax.experimental.pallas import tpu as pltpu
from jax.experimental.pallas import tpu_sc as plsc

N_NODES = 10000
N_EDGES = 320000
D = 128

NC = 2
NS = 16
NW = NC * NS
LANES = 16

EPW = N_EDGES // NW
C = 80
NCHUNK = EPW // C
N_PAD = 10240
RPS = N_PAD // NS
ZR = 128


def _tanh_body(x_ref, o_ref):
    o_ref[...] = jnp.tanh(x_ref[...])


def _tanh_tc(x):
    return pl.pallas_call(
        _tanh_body,
        out_shape=jax.ShapeDtypeStruct((N_NODES, D), jnp.float32),
        grid=(10,),
        in_specs=[pl.BlockSpec((N_NODES // 10, D), lambda i: (i, 0))],
        out_specs=pl.BlockSpec((N_NODES // 10, D), lambda i: (i, 0)),
    )(x)


def _add_body(p_ref, o_ref):
    o_ref[...] = p_ref[0] + p_ref[1]


def _add_tc(partials):
    return pl.pallas_call(
        _add_body,
        out_shape=jax.ShapeDtypeStruct((N_NODES, D), jnp.float32),
        grid=(10,),
        in_specs=[pl.BlockSpec((NC, N_NODES // 10, D), lambda i: (0, i, 0))],

        out_specs=pl.BlockSpec((N_NODES // 10, D), lambda i: (i, 0)),
    )(partials)


def _sc_scatter(t, src, dst, w):
    mesh = plsc.VectorSubcoreMesh(
        core_axis_name="c", subcore_axis_name="s", num_cores=NC,
        num_subcores=NS)

    @functools.partial(
        pl.kernel,
        mesh=mesh,
        out_type=jax.ShapeDtypeStruct((NC, N_PAD, D), jnp.float32),
        scratch_types=[
            pltpu.VMEM((C,), jnp.int32),
            pltpu.VMEM((C,), jnp.int32),
            pltpu.VMEM((C,), jnp.float32),
            pltpu.VMEM((C, D), jnp.float32),
            pltpu.VMEM((ZR, D), jnp.float32),
            pltpu.VMEM_SHARED((N_PAD, D), jnp.float32),
            pltpu.SemaphoreType.DMA,
        ],
    )
    def k(t_hbm, src_hbm, dst_hbm, w_hbm, out_hbm,
          dst_v, src_v, w_v, rows_v, zbuf, acc, sem):
        cid = lax.axis_index("c")
        sid = lax.axis_index("s")
        wid = sid * NC + cid

        def zrow(i, _):
            for kk in range(D // LANES):
                zbuf[i, pl.ds(kk * LANES, LANES)] = jnp.zeros(
                    (LANES,), jnp.float32)
            return _
        lax.fori_loop(0, ZR, zrow, 0)
        row0 = sid * RPS
        for j in range(RPS // ZR):
            pltpu.sync_copy(zbuf, acc.at[pl.ds(row0 + j * ZR, ZR)])
        plsc.subcore_barrier()

        def chunk(j, _):
            base = wid * EPW + j * C
            pltpu.sync_copy(dst_hbm.at[pl.ds(base, C)], dst_v)
            pltpu.sync_copy(src_hbm.at[pl.ds(base, C)], src_v)
            pltpu.sync_copy(w_hbm.at[pl.ds(base, C)], w_v)
            pltpu.async_copy(t_hbm.at[dst_v], rows_v, sem).wait()

            def scale(g, _):
                wvec = w_v[pl.ds(g * LANES, LANES)]
                for l in range(LANES):
                    ws = wvec[l]
                    e = g * LANES + l
                    for kk in range(D // LANES):
                        sl = pl.ds(kk * LANES, LANES)
                        rows_v[e, sl] = rows_v[e, sl] * ws
                return _
            lax.fori_loop(0, C // LANES, scale, 0)
            pltpu.sync_copy(rows_v, acc.at[src_v], add=True)
            return _
        lax.fori_loop(0, NCHUNK, chunk, 0)
        plsc.subcore_barrier()

        for j in range(RPS // ZR):
            r = row0 + j * ZR
            pltpu.sync_copy(acc.at[pl.ds(r, ZR)], zbuf)
            pltpu.sync_copy(zbuf, out_hbm.at[cid, pl.ds(r, ZR)])

    return k(t, src, dst, w)


def kernel(x, edge_index, weight):
    src = edge_index[0].astype(jnp.int32)
    dst = edge_index[1].astype(jnp.int32)
    t = _tanh_tc(x)
    partials = _sc_scatter(t, src, dst, weight)
    return _add_tc(partials)


# speedup vs baseline: 8.5102x; 1.9124x over previous
import functools

import jax
import jax.numpy as jnp
from jax import lax
from jax.experimental import pallas as pl
from jax.experimental.pallas import tpu as pltpu
from jax.experimental.pallas import tpu_sc as plsc

N_NODES = 10000
N_EDGES = 320000
D = 128

NC = 2
NS = 16
NW = NC * NS
LANES = 16

EPW = N_EDGES // NW
C = 80
NCHUNK = EPW // C
N_PAD = 10240
RPS = N_PAD // NS
ZR = 64


def _tanh_body(x_ref, o_ref):
    o_ref[...] = jnp.tanh(x_ref[...])


def _tanh_tc(x):
    return pl.pallas_call(
        _tanh_body,
        out_shape=jax.ShapeDtypeStruct((N_NODES, D), jnp.float32),
        grid=(10,),
        in_specs=[pl.BlockSpec((N_NODES // 10, D), lambda i: (i, 0))],
        out_specs=pl.BlockSpec((N_NODES // 10, D), lambda i: (i, 0)),
    )(x)


def _add_body(p_ref, o_ref):
    o_ref[...] = p_ref[0] + p_ref[1]


def _add_tc(partials):
    return pl.pallas_call(
        _add_body,
        out_shape=jax.ShapeDtypeStruct((N_NODES, D), jnp.float32),
        grid=(10,),
        in_specs=[pl.BlockSpec((NC, N_NODES // 10, D), lambda i: (0, i, 0))],

        out_specs=pl.BlockSpec((N_NODES // 10, D), lambda i: (i, 0)),
    )(partials)


def _sc_scatter(t, packed, w):
    mesh = plsc.VectorSubcoreMesh(
        core_axis_name="c", subcore_axis_name="s", num_cores=NC,
        num_subcores=NS)

    @functools.partial(
        pl.kernel,
        mesh=mesh,
        out_type=jax.ShapeDtypeStruct((NC, N_PAD, D), jnp.float32),
        scratch_types=[
            pltpu.VMEM((2, C), jnp.int32),
            pltpu.VMEM((2, C), jnp.int32),
            pltpu.VMEM((C,), jnp.float32),
            pltpu.VMEM((C,), jnp.float32),
            pltpu.VMEM((C, D), jnp.float32),
            pltpu.VMEM((C, D), jnp.float32),
            pltpu.VMEM((C,), jnp.int32),
            pltpu.VMEM((ZR, D), jnp.float32),
            pltpu.VMEM_SHARED((N_PAD, D), jnp.float32),
            pltpu.SemaphoreType.DMA,
            pltpu.SemaphoreType.DMA,
            pltpu.SemaphoreType.DMA,
        ],
    )
    def k(t_hbm, pk_hbm, w_hbm, out_hbm,
          e0, e1, w0, w1, buf_a, buf_b, src_sm, zbuf, acc,
          sem_a, sem_b, sem_i):
        cid = lax.axis_index("c")
        sid = lax.axis_index("s")
        wid = sid * NC + cid
        cbase = wid * NCHUNK

        def zrow(i, _):
            for kk in range(D // LANES):
                zbuf[i, pl.ds(kk * LANES, LANES)] = jnp.zeros(
                    (LANES,), jnp.float32)
            return _
        lax.fori_loop(0, ZR, zrow, 0)
        row0 = sid * RPS
        for j in range(RPS // ZR):
            pltpu.sync_copy(zbuf, acc.at[pl.ds(row0 + j * ZR, ZR)])
        plsc.subcore_barrier()

        def idx_load(j, ebuf, sem):
            return pltpu.make_async_copy(pk_hbm.at[cbase + j], ebuf, sem)

        def w_load(j, wbuf, sem):
            return pltpu.make_async_copy(
                w_hbm.at[pl.ds((cbase + j) * C, C)], wbuf, sem)

        def gather(ebuf, buf, sem):
            return pltpu.make_async_copy(t_hbm.at[ebuf.at[1]], buf, sem)

        def consume(j, ebuf, wbuf, buf):
            def scale(g, _):
                wvec = wbuf[pl.ds(g * LANES, LANES)]
                for l in range(LANES):
                    ws = wvec[l]
                    e = g * LANES + l
                    for kk in range(D // LANES):
                        sl = pl.ds(kk * LANES, LANES)
                        buf[e, sl] = buf[e, sl] * ws
                return _
            lax.fori_loop(0, C // LANES, scale, 0)
            for g in range(C // LANES):
                sl = pl.ds(g * LANES, LANES)
                src_sm[sl] = ebuf[0, sl]
            pltpu.sync_copy(buf, acc.at[src_sm], add=True)

        idx_load(0, e0, sem_i).start()
        w_load(0, w0, sem_i).start()
        idx_load(0, e0, sem_i).wait()
        w_load(0, w0, sem_i).wait()
        gather(e0, buf_a, sem_a).start()
        idx_load(1, e1, sem_i).start()
        w_load(1, w1, sem_i).start()

        def pair(p, _):
            j0 = 2 * p
            idx_load(j0 + 1, e1, sem_i).wait()
            w_load(j0 + 1, w1, sem_i).wait()
            gather(e1, buf_b, sem_b).start()
            gather(e0, buf_a, sem_a).wait()
            consume(j0, e0, w0, buf_a)
            idx_load(j0 + 2, e0, sem_i).start()
            w_load(j0 + 2, w0, sem_i).start()
            idx_load(j0 + 2, e0, sem_i).wait()
            w_load(j0 + 2, w0, sem_i).wait()
            gather(e0, buf_a, sem_a).start()
            gather(e1, buf_b, sem_b).wait()
            consume(j0 + 1, e1, w1, buf_b)

            @pl.when(j0 + 3 < NCHUNK)
            def _start_next():
                idx_load(j0 + 3, e1, sem_i).start()
                w_load(j0 + 3, w1, sem_i).start()
            return _
        lax.fori_loop(0, (NCHUNK - 1) // 2, pair, 0)
        gather(e0, buf_a, sem_a).wait()
        consume(NCHUNK - 1, e0, w0, buf_a)
        plsc.subcore_barrier()

        for j in range(RPS // ZR):
            r = row0 + j * ZR
            pltpu.sync_copy(acc.at[pl.ds(r, ZR)], zbuf)
            pltpu.sync_copy(zbuf, out_hbm.at[cid, pl.ds(r, ZR)])

    return k(t, packed, w)


def kernel(x, edge_index, weight):
    src = edge_index[0].astype(jnp.int32)
    dst = edge_index[1].astype(jnp.int32)
    nct = N_EDGES // C
    packed = jnp.stack(
        [src.reshape(nct, C), dst.reshape(nct, C)],
        axis=1)
    t = _tanh_tc(x)
    partials = _sc_scatter(t, packed, weight)
    return _add_tc(partials)


# speedup vs baseline: 11.4103x; 1.3408x over previous
import functools

import jax
import jax.numpy as jnp
from jax import lax
from jax.experimental import pallas as pl
from jax.experimental.pallas import tpu as pltpu
from jax.experimental.pallas import tpu_sc as plsc

N_NODES = 10000
N_EDGES = 320000
D = 128

NC = 2
NS = 16
NW = NC * NS
LANES = 16

EPW = N_EDGES // NW
C = 80
NCHUNK = EPW // C
N_PAD = 10240
RPS = N_PAD // NS
ZR = 64


def _tanh_body(x_ref, o_ref):
    o_ref[...] = jnp.tanh(x_ref[...])


def _tanh_tc(x):
    return pl.pallas_call(
        _tanh_body,
        out_shape=jax.ShapeDtypeStruct((N_NODES, D), jnp.float32),
        grid=(10,),
        in_specs=[pl.BlockSpec((N_NODES // 10, D), lambda i: (i, 0))],
        out_specs=pl.BlockSpec((N_NODES // 10, D), lambda i: (i, 0)),
    )(x)


def _add_body(p_ref, o_ref):
    o_ref[...] = p_ref[0] + p_ref[1]


def _add_tc(partials):
    return pl.pallas_call(
        _add_body,
        out_shape=jax.ShapeDtypeStruct((N_NODES, D), jnp.float32),
        grid=(10,),
        in_specs=[pl.BlockSpec((NC, N_NODES // 10, D), lambda i: (0, i, 0))],

        out_specs=pl.BlockSpec((N_NODES // 10, D), lambda i: (i, 0)),
    )(partials)


def _sc_scatter(t, packed, w):
    mesh = plsc.VectorSubcoreMesh(
        core_axis_name="c", subcore_axis_name="s", num_cores=NC,
        num_subcores=NS)

    @functools.partial(
        pl.kernel,
        mesh=mesh,
        out_type=jax.ShapeDtypeStruct((NC, N_PAD, D), jnp.float32),
        scratch_types=[
            [pltpu.VMEM((2, C), jnp.int32) for _ in range(3)],
            [pltpu.VMEM((C,), jnp.float32) for _ in range(3)],
            [pltpu.VMEM((C, D), jnp.float32) for _ in range(3)],
            [pltpu.VMEM((C,), jnp.int32) for _ in range(3)],
            pltpu.VMEM((ZR, D), jnp.float32),
            pltpu.VMEM_SHARED((N_PAD, D), jnp.float32),
            [pltpu.SemaphoreType.DMA for _ in range(3)],
            [pltpu.SemaphoreType.DMA for _ in range(3)],
            pltpu.SemaphoreType.DMA,
        ],
    )
    def k(t_hbm, pk_hbm, w_hbm, out_hbm,
          ebufs, wbufs, bufs, srcs, zbuf, acc, gsems, ssems, sem_i):
        cid = lax.axis_index("c")
        sid = lax.axis_index("s")
        wid = sid * NC + cid
        cbase = wid * NCHUNK

        def zrow(i, _):
            for kk in range(D // LANES):
                zbuf[i, pl.ds(kk * LANES, LANES)] = jnp.zeros(
                    (LANES,), jnp.float32)
            return _
        lax.fori_loop(0, ZR, zrow, 0)
        row0 = sid * RPS
        for j in range(RPS // ZR):
            pltpu.sync_copy(zbuf, acc.at[pl.ds(row0 + j * ZR, ZR)])
        plsc.subcore_barrier()

        def idx_start(j, r):
            pltpu.make_async_copy(pk_hbm.at[cbase + j], ebufs[r], sem_i).start()
            pltpu.make_async_copy(
                w_hbm.at[pl.ds((cbase + j) * C, C)], wbufs[r], sem_i).start()

        def idx_wait(r):
            pltpu.make_async_copy(pk_hbm.at[cbase], ebufs[r], sem_i).wait()
            pltpu.make_async_copy(
                w_hbm.at[pl.ds(cbase * C, C)], wbufs[r], sem_i).wait()

        def gather(j, r):
            return pltpu.make_async_copy(
                t_hbm.at[ebufs[r].at[1]], bufs[r], gsems[r])

        def scatter(r):
            return pltpu.make_async_copy(
                bufs[r], acc.at[srcs[r]], ssems[r])

        def scale_and_scatter(r):
            buf, wbuf, ebuf = bufs[r], wbufs[r], ebufs[r]

            def scale(g, _):
                wvec = wbuf[pl.ds(g * LANES, LANES)]
                for l in range(LANES):
                    ws = wvec[l]
                    e = g * LANES + l
                    for kk in range(D // LANES):
                        sl = pl.ds(kk * LANES, LANES)
                        buf[e, sl] = buf[e, sl] * ws
                return _
            lax.fori_loop(0, C // LANES, scale, 0)
            for g in range(C // LANES):
                sl = pl.ds(g * LANES, LANES)
                srcs[r][sl] = ebuf[0, sl]
            scatter(r).start(add=True)

        idx_start(0, 0)
        idx_wait(0)
        gather(0, 0).start()
        idx_start(1, 1)

        def step(j, cs, first):
            ns = (cs + 1) % 3
            idx_wait(ns)
            if first:
                pass
            else:
                scatter(ns).wait()
            gather(j + 1, ns).start()
            idx_start(j + 2, (cs + 2) % 3)
            gather(j, cs).wait()
            scale_and_scatter(cs)

        step(0, 0, True)
        step(1, 1, True)
        step(2, 2, False)

        def triple(q, _):
            j0 = 3 * q
            step(j0, 0, False)
            step(j0 + 1, 1, False)
            step(j0 + 2, 2, False)
            return _
        lax.fori_loop(1, (NCHUNK - 2) // 3, triple, 0)

        idx_wait(1)
        scatter(1).wait()
        gather(NCHUNK - 1, 1).start()
        gather(NCHUNK - 2, 0).wait()
        scale_and_scatter(0)
        gather(NCHUNK - 1, 1).wait()
        scale_and_scatter(1)
        scatter(2).wait()
        scatter(0).wait()
        scatter(1).wait()
        plsc.subcore_barrier()

        for j in range(RPS // ZR):
            r = row0 + j * ZR
            pltpu.sync_copy(acc.at[pl.ds(r, ZR)], zbuf)
            pltpu.sync_copy(zbuf, out_hbm.at[cid, pl.ds(r, ZR)])

    return k(t, packed, w)


def kernel(x, edge_index, weight):
    src = edge_index[0].astype(jnp.int32)
    dst = edge_index[1].astype(jnp.int32)
    nct = N_EDGES // C
    packed = jnp.stack(
        [src.reshape(nct, C), dst.reshape(nct, C)],
        axis=1)
    t = _tanh_tc(x)
    partials = _sc_scatter(t, packed, weight)
    return _add_tc(partials)
